# Initial kernel scaffold; baseline (speedup 1.0000x reference)
import functools

import jax
import jax.numpy as jnp
from jax import lax
from jax.experimental import pallas as pl
from jax.experimental.pallas import tpu as pltpu
from jax.experimental.pallas import tpu_sc as plsc

N_NODES = 10000
D = 128
E = 320000
NS = 16
E_TILE = E // NS
CS = 80
NCHUNK = E_TILE // CS
ROWS_TILE = N_NODES // NS
ZROWS = 125


def _zero_acc(acc, zbuf, s):
    def zrow(i, carry):
        for k in range(D // 16):
            zbuf[i, pl.ds(k * 16, 16)] = jnp.zeros((16,), jnp.float32)
        return carry

    lax.fori_loop(0, ZROWS, zrow, 0)
    for r in range(ROWS_TILE // ZROWS):
        pltpu.sync_copy(
            zbuf, acc.at[pl.ds(s * ROWS_TILE + r * ZROWS, ZROWS)]
        )


def _hop(x_hbm, dst_hbm, src_hbm, idx_d, idx_s, rows, acc, sem, s):
    pltpu.sync_copy(dst_hbm.at[pl.ds(s * NCHUNK, NCHUNK)], idx_d)
    pltpu.sync_copy(src_hbm.at[pl.ds(s * NCHUNK, NCHUNK)], idx_s)

    def chunk(i, carry):
        pltpu.async_copy(x_hbm.at[idx_s.at[i]], rows, sem).wait()
        pltpu.sync_copy(rows, acc.at[idx_d.at[i]], add=True)
        return carry

    lax.fori_loop(0, NCHUNK, chunk, 0)


def _body(x_hbm, d1, s1, d2, s2, out_hbm, idx_d, idx_s, rows, acc, zbuf, sem):
    c = lax.axis_index("c")
    s = lax.axis_index("s")

    _zero_acc(acc, zbuf, s)
    plsc.subcore_barrier()

    @pl.when(c == 0)
    def _():
        _hop(x_hbm, d1, s1, idx_d, idx_s, rows, acc, sem, s)

    @pl.when(c == 1)
    def _():
        _hop(x_hbm, d2, s2, idx_d, idx_s, rows, acc, sem, s)

    plsc.subcore_barrier()

    @pl.when(c == 0)
    def _():
        pltpu.sync_copy(
            acc.at[pl.ds(s * ROWS_TILE, ROWS_TILE)],
            out_hbm.at[pl.ds(s * ROWS_TILE, ROWS_TILE), 0:D],
        )

    @pl.when(c == 1)
    def _():
        pltpu.sync_copy(
            acc.at[pl.ds(s * ROWS_TILE, ROWS_TILE)],
            out_hbm.at[pl.ds(s * ROWS_TILE, ROWS_TILE), D : 2 * D],
        )


_sc_kernel = functools.partial(
    pl.kernel,
    mesh=plsc.VectorSubcoreMesh(core_axis_name="c", subcore_axis_name="s"),
    out_type=jax.ShapeDtypeStruct((N_NODES, 2 * D), jnp.float32),
    scratch_types=[
        pltpu.VMEM((NCHUNK, CS), jnp.int32),
        pltpu.VMEM((NCHUNK, CS), jnp.int32),
        pltpu.VMEM((CS, D), jnp.float32),
        pltpu.VMEM((ZROWS, D), jnp.float32),
        pltpu.VMEM_SHARED((N_NODES, D), jnp.float32),
        pltpu.SemaphoreType.DMA,
    ],
)(_body)


@jax.jit
def kernel(x, adj_t, adj_t2):
    d1 = adj_t[0].reshape(NS * NCHUNK, CS)
    s1 = adj_t[1].reshape(NS * NCHUNK, CS)
    d2 = adj_t2[0].reshape(NS * NCHUNK, CS)
    s2 = adj_t2[1].reshape(NS * NCHUNK, CS)
    return _sc_kernel(x, d1, s1, d2, s2)


# speedup vs baseline: 2.5716x; 2.5716x over previous
import functools

import jax
import jax.numpy as jnp
from jax import lax
from jax.experimental import pallas as pl
from jax.experimental.pallas import tpu as pltpu
from jax.experimental.pallas import tpu_sc as plsc

N_NODES = 10000
D = 128
DH = D // 2
E = 320000
NS = 16
CS = 80
NCHUNK = E // NS // CS
NCHUNKP = 256
GARB = N_NODES
ACC_ROWS = 10016
ZB = 16
WB = 80


def _hop_pass(x_hbm, o_hbm, idx_d, idx_s, rows, zbuf, acc, sem, s):
    def zb(j, carry):
        pltpu.sync_copy(zbuf, acc.at[pl.ds((s + j * NS) * ZB, ZB)])
        return carry

    lax.fori_loop(0, (641 - s) // 16, zb, 0)
    plsc.subcore_barrier()

    def chunk(i, carry):
        pltpu.async_copy(x_hbm.at[idx_s.at[i]], rows, sem).wait()
        pltpu.sync_copy(rows, acc.at[idx_d.at[i]], add=True)
        return carry

    lax.fori_loop(0, NCHUNKP, chunk, 0)
    plsc.subcore_barrier()

    def wb(j, carry):
        r0 = (s + j * NS) * WB
        pltpu.sync_copy(acc.at[pl.ds(r0, WB)], o_hbm.at[pl.ds(r0, WB)])
        return carry

    lax.fori_loop(0, (140 - s) // 16, wb, 0)
    plsc.subcore_barrier()


def _body(xa, xb, d1, s1, d2, s2, o1a, o1b, o2a, o2b,
          idx_d, idx_s, rows, zbuf, acc, sem):
    c = lax.axis_index("c")
    s = lax.axis_index("s")

    def zrow(i, carry):
        for k in range(DH // 16):
            zbuf[i, pl.ds(k * 16, 16)] = jnp.zeros((16,), jnp.float32)
        return carry

    lax.fori_loop(0, ZB, zrow, 0)

    @pl.when(c == 0)
    def _():
        pltpu.sync_copy(d1.at[pl.ds(s * NCHUNKP, NCHUNKP)], idx_d)
        pltpu.sync_copy(s1.at[pl.ds(s * NCHUNKP, NCHUNKP)], idx_s)
        _hop_pass(xa, o1a, idx_d, idx_s, rows, zbuf, acc, sem, s)
        _hop_pass(xb, o1b, idx_d, idx_s, rows, zbuf, acc, sem, s)

    @pl.when(c == 1)
    def _():
        pltpu.sync_copy(d2.at[pl.ds(s * NCHUNKP, NCHUNKP)], idx_d)
        pltpu.sync_copy(s2.at[pl.ds(s * NCHUNKP, NCHUNKP)], idx_s)
        _hop_pass(xa, o2a, idx_d, idx_s, rows, zbuf, acc, sem, s)
        _hop_pass(xb, o2b, idx_d, idx_s, rows, zbuf, acc, sem, s)


_half_out = jax.ShapeDtypeStruct((N_NODES, DH), jnp.float32)

_sc_kernel = functools.partial(
    pl.kernel,
    mesh=plsc.VectorSubcoreMesh(core_axis_name="c", subcore_axis_name="s"),
    out_type=[_half_out, _half_out, _half_out, _half_out],
    compiler_params=pltpu.CompilerParams(use_tc_tiling_on_sc=False),
    scratch_types=[
        pltpu.VMEM((NCHUNKP, CS), jnp.int32),
        pltpu.VMEM((NCHUNKP, CS), jnp.int32),
        pltpu.VMEM((CS, DH), jnp.float32),
        pltpu.VMEM((ZB, DH), jnp.float32),
        pltpu.VMEM_SHARED((ACC_ROWS, DH), jnp.float32),
        pltpu.SemaphoreType.DMA,
    ],
)(_body)


def _pad_idx(row, pad_val):
    a = row.reshape(NS, NCHUNK, CS)
    pad = jnp.full((NS, NCHUNKP - NCHUNK, CS), pad_val, jnp.int32)
    return jnp.concatenate([a, pad], axis=1).reshape(NS * NCHUNKP, CS)


@jax.jit
def kernel(x, adj_t, adj_t2):
    xa = x[:, :DH]
    xb = x[:, DH:]
    d1 = _pad_idx(adj_t[0], GARB)
    s1 = _pad_idx(adj_t[1], 0)
    d2 = _pad_idx(adj_t2[0], GARB)
    s2 = _pad_idx(adj_t2[1], 0)
    o1a, o1b, o2a, o2b = _sc_kernel(xa, xb, d1, s1, d2, s2)
    return jnp.concatenate([o1a, o1b, o2a, o2b], axis=1)


# speedup vs baseline: 3.5942x; 1.3976x over previous
import functools

import jax
import jax.numpy as jnp
from jax import lax
from jax.experimental import pallas as pl
from jax.experimental.pallas import tpu as pltpu
from jax.experimental.pallas import tpu_sc as plsc

N_NODES = 10000
D = 128
DH = D // 2
E = 320000
NS = 16
CS = 128
NCHUNKP = 160
EPAD = NS * NCHUNKP * CS
GARB = N_NODES
ACC_ROWS = 10016
ZB = 16
WB = 80


def _hop_pass(x_hbm, o_hbm, idx_d, idx_s, r0buf, r1buf, zbuf, acc, sem0, sem1, s):
    def zb(j, carry):
        pltpu.sync_copy(zbuf, acc.at[pl.ds((s + j * NS) * ZB, ZB)])
        return carry

    lax.fori_loop(0, (641 - s) // 16, zb, 0)
    plsc.subcore_barrier()

    pltpu.async_copy(x_hbm.at[idx_s.at[0]], r0buf, sem0)

    def pair(j, carry):
        i0 = 2 * j
        pltpu.async_copy(x_hbm.at[idx_s.at[i0 + 1]], r1buf, sem1)
        pltpu.make_async_copy(x_hbm.at[idx_s.at[i0]], r0buf, sem0).wait()
        pltpu.sync_copy(r0buf, acc.at[idx_d.at[i0]], add=True)

        @pl.when(i0 + 2 < NCHUNKP)
        def _():
            pltpu.async_copy(x_hbm.at[idx_s.at[i0 + 2]], r0buf, sem0)

        pltpu.make_async_copy(x_hbm.at[idx_s.at[i0 + 1]], r1buf, sem1).wait()
        pltpu.sync_copy(r1buf, acc.at[idx_d.at[i0 + 1]], add=True)
        return carry

    lax.fori_loop(0, NCHUNKP // 2, pair, 0)
    plsc.subcore_barrier()

    def wb(j, carry):
        r0 = (s + j * NS) * WB
        pltpu.sync_copy(acc.at[pl.ds(r0, WB)], o_hbm.at[pl.ds(r0, WB)])
        return carry

    lax.fori_loop(0, (140 - s) // 16, wb, 0)
    plsc.subcore_barrier()


def _body(xa, xb, d1, s1, d2, s2, o1a, o1b, o2a, o2b,
          idx_d, idx_s, r0buf, r1buf, zbuf, acc, sem0, sem1):
    c = lax.axis_index("c")
    s = lax.axis_index("s")

    def zrow(i, carry):
        for k in range(DH // 16):
            zbuf[i, pl.ds(k * 16, 16)] = jnp.zeros((16,), jnp.float32)
        return carry

    lax.fori_loop(0, ZB, zrow, 0)

    @pl.when(c == 0)
    def _():
        pltpu.sync_copy(d1.at[pl.ds(s * NCHUNKP, NCHUNKP)], idx_d)
        pltpu.sync_copy(s1.at[pl.ds(s * NCHUNKP, NCHUNKP)], idx_s)
        _hop_pass(xa, o1a, idx_d, idx_s, r0buf, r1buf, zbuf, acc, sem0, sem1, s)
        _hop_pass(xb, o1b, idx_d, idx_s, r0buf, r1buf, zbuf, acc, sem0, sem1, s)

    @pl.when(c == 1)
    def _():
        pltpu.sync_copy(d2.at[pl.ds(s * NCHUNKP, NCHUNKP)], idx_d)
        pltpu.sync_copy(s2.at[pl.ds(s * NCHUNKP, NCHUNKP)], idx_s)
        _hop_pass(xa, o2a, idx_d, idx_s, r0buf, r1buf, zbuf, acc, sem0, sem1, s)
        _hop_pass(xb, o2b, idx_d, idx_s, r0buf, r1buf, zbuf, acc, sem0, sem1, s)


_half_out = jax.ShapeDtypeStruct((N_NODES, DH), jnp.float32)

_sc_kernel = functools.partial(
    pl.kernel,
    mesh=plsc.VectorSubcoreMesh(core_axis_name="c", subcore_axis_name="s"),
    out_type=[_half_out, _half_out, _half_out, _half_out],
    compiler_params=pltpu.CompilerParams(use_tc_tiling_on_sc=False),
    scratch_types=[
        pltpu.VMEM((NCHUNKP, CS), jnp.int32),
        pltpu.VMEM((NCHUNKP, CS), jnp.int32),
        pltpu.VMEM((CS, DH), jnp.float32),
        pltpu.VMEM((CS, DH), jnp.float32),
        pltpu.VMEM((ZB, DH), jnp.float32),
        pltpu.VMEM_SHARED((ACC_ROWS, DH), jnp.float32),
        pltpu.SemaphoreType.DMA,
        pltpu.SemaphoreType.DMA,
    ],
)(_body)


def _pad_idx(row, pad_val):
    pad = jnp.full((EPAD - E,), pad_val, jnp.int32)
    return jnp.concatenate([row, pad]).reshape(NS * NCHUNKP, CS)


@jax.jit
def kernel(x, adj_t, adj_t2):
    xa = x[:, :DH]
    xb = x[:, DH:]
    d1 = _pad_idx(adj_t[0], GARB)
    s1 = _pad_idx(adj_t[1], 0)
    d2 = _pad_idx(adj_t2[0], GARB)
    s2 = _pad_idx(adj_t2[1], 0)
    o1a, o1b, o2a, o2b = _sc_kernel(xa, xb, d1, s1, d2, s2)
    return jnp.concatenate([o1a, o1b, o2a, o2b], axis=1)


# speedup vs baseline: 3.7254x; 1.0365x over previous
import functools

import jax
import jax.numpy as jnp
from jax import lax
from jax.experimental import pallas as pl
from jax.experimental.pallas import tpu as pltpu
from jax.experimental.pallas import tpu_sc as plsc

N_NODES = 10000
D = 128
DH = D // 2
E = 320000
NS = 16
CS = 128
NCHUNKP = 160
EPAD = NS * NCHUNKP * CS
GARB = N_NODES
ACC_ROWS = 10016
ZB = 16
WB = 80
NB = 4


def _hop_pass(x_hbm, o_hbm, idx_d, idx_s, rbufs, zbuf, acc, gsems, csems, s):
    def zblk(j, carry):
        pltpu.sync_copy(zbuf, acc.at[pl.ds((s + j * NS) * ZB, ZB)])
        return carry

    lax.fori_loop(0, (641 - s) // 16, zblk, 0)
    plsc.subcore_barrier()

    def fire_gather(i, b):
        pltpu.async_copy(x_hbm.at[idx_s.at[i]], rbufs[b], gsems[b])

    def wait_gather(i, b):
        pltpu.make_async_copy(x_hbm.at[idx_s.at[i]], rbufs[b], gsems[b]).wait()

    def fire_scatter(i, b):
        pltpu.async_copy(rbufs[b], acc.at[idx_d.at[i]], csems[b], add=True)

    def wait_scatter(i, b):
        pltpu.make_async_copy(rbufs[b], acc.at[idx_d.at[i]], csems[b]).wait()

    fire_gather(0, 0)
    fire_gather(1, 1)
    fire_gather(2, 2)
    wait_gather(0, 0)
    fire_scatter(0, 0)
    fire_gather(3, 3)
    wait_gather(1, 1)
    fire_scatter(1, 1)
    wait_scatter(0, 0)
    fire_gather(4, 0)
    wait_gather(2, 2)
    fire_scatter(2, 2)
    wait_scatter(1, 1)
    fire_gather(5, 1)
    wait_gather(3, 3)
    fire_scatter(3, 3)

    def group(g, carry):
        i0 = 4 * g
        for b in range(NB):
            i = i0 + b
            nb = (b + 2) % NB

            @pl.when(i + 2 < NCHUNKP)
            def _():
                wait_scatter(i - 2, nb)
                fire_gather(i + 2, nb)

            wait_gather(i, b)
            fire_scatter(i, b)
        return carry

    lax.fori_loop(1, NCHUNKP // NB, group, 0)

    for i in range(NCHUNKP - NB, NCHUNKP):
        wait_scatter(i, i % NB)
    plsc.subcore_barrier()

    def wblk(j, carry):
        r0 = (s + j * NS) * WB
        pltpu.sync_copy(acc.at[pl.ds(r0, WB)], o_hbm.at[pl.ds(r0, WB)])
        return carry

    lax.fori_loop(0, (140 - s) // 16, wblk, 0)
    plsc.subcore_barrier()


def _body(xa, xb, d1, s1, d2, s2, o1a, o1b, o2a, o2b,
          idx_d, idx_s, r0, r1, r2, r3, zbuf, acc,
          g0, g1, g2, g3, c0, c1, c2, c3):
    c = lax.axis_index("c")
    s = lax.axis_index("s")
    rbufs = (r0, r1, r2, r3)
    gsems = (g0, g1, g2, g3)
    csems = (c0, c1, c2, c3)

    def zrow(i, carry):
        for k in range(DH // 16):
            zbuf[i, pl.ds(k * 16, 16)] = jnp.zeros((16,), jnp.float32)
        return carry

    lax.fori_loop(0, ZB, zrow, 0)

    @pl.when(c == 0)
    def _():
        pltpu.sync_copy(d1.at[pl.ds(s * NCHUNKP, NCHUNKP)], idx_d)
        pltpu.sync_copy(s1.at[pl.ds(s * NCHUNKP, NCHUNKP)], idx_s)
        _hop_pass(xa, o1a, idx_d, idx_s, rbufs, zbuf, acc, gsems, csems, s)
        _hop_pass(xb, o1b, idx_d, idx_s, rbufs, zbuf, acc, gsems, csems, s)

    @pl.when(c == 1)
    def _():
        pltpu.sync_copy(d2.at[pl.ds(s * NCHUNKP, NCHUNKP)], idx_d)
        pltpu.sync_copy(s2.at[pl.ds(s * NCHUNKP, NCHUNKP)], idx_s)
        _hop_pass(xa, o2a, idx_d, idx_s, rbufs, zbuf, acc, gsems, csems, s)
        _hop_pass(xb, o2b, idx_d, idx_s, rbufs, zbuf, acc, gsems, csems, s)


_half_out = jax.ShapeDtypeStruct((N_NODES, DH), jnp.float32)

_sc_kernel = functools.partial(
    pl.kernel,
    mesh=plsc.VectorSubcoreMesh(core_axis_name="c", subcore_axis_name="s"),
    out_type=[_half_out, _half_out, _half_out, _half_out],
    compiler_params=pltpu.CompilerParams(use_tc_tiling_on_sc=False),
    scratch_types=[
        pltpu.VMEM((NCHUNKP, CS), jnp.int32),
        pltpu.VMEM((NCHUNKP, CS), jnp.int32),
        pltpu.VMEM((CS, DH), jnp.float32),
        pltpu.VMEM((CS, DH), jnp.float32),
        pltpu.VMEM((CS, DH), jnp.float32),
        pltpu.VMEM((CS, DH), jnp.float32),
        pltpu.VMEM((ZB, DH), jnp.float32),
        pltpu.VMEM_SHARED((ACC_ROWS, DH), jnp.float32),
        pltpu.SemaphoreType.DMA,
        pltpu.SemaphoreType.DMA,
        pltpu.SemaphoreType.DMA,
        pltpu.SemaphoreType.DMA,
        pltpu.SemaphoreType.DMA,
        pltpu.SemaphoreType.DMA,
        pltpu.SemaphoreType.DMA,
        pltpu.SemaphoreType.DMA,
    ],
)(_body)


def _pad_idx(row, pad_val):
    pad = jnp.full((EPAD - E,), pad_val, jnp.int32)
    return jnp.concatenate([row, pad]).reshape(NS * NCHUNKP, CS)


@jax.jit
def kernel(x, adj_t, adj_t2):
    xa = x[:, :DH]
    xb = x[:, DH:]
    d1 = _pad_idx(adj_t[0], GARB)
    s1 = _pad_idx(adj_t[1], 0)
    d2 = _pad_idx(adj_t2[0], GARB)
    s2 = _pad_idx(adj_t2[1], 0)
    o1a, o1b, o2a, o2b = _sc_kernel(xa, xb, d1, s1, d2, s2)
    return jnp.concatenate([o1a, o1b, o2a, o2b], axis=1)


# speedup vs baseline: 3.7352x; 1.0026x over previous
import functools

import jax
import jax.numpy as jnp
from jax import lax
from jax.experimental import pallas as pl
from jax.experimental.pallas import tpu as pltpu
from jax.experimental.pallas import tpu_sc as plsc

N_NODES = 10000
D = 128
DH = D // 2
E = 320000
NS = 16
CS = 128
NCHUNKP = 160
EPAD = NS * NCHUNKP * CS
GARB = N_NODES
ACC_ROWS = 10016
ZB = 16
WB = 80
NB = 4


def _hop_pass(x_hbm, o_hbm, idx_d, idx_s, rbufs, zbuf, acc, gsems, csems, s):
    def zblk(j, carry):
        pltpu.sync_copy(zbuf, acc.at[pl.ds((s + j * NS) * ZB, ZB)])
        return carry

    lax.fori_loop(0, (641 - s) // 16, zblk, 0)
    plsc.subcore_barrier()

    HC = CS // 2

    def fire_gather(i, b):
        pltpu.async_copy(
            x_hbm.at[idx_s.at[i, pl.ds(0, HC)]],
            rbufs[b].at[pl.ds(0, HC)], gsems[b])
        pltpu.async_copy(
            x_hbm.at[idx_s.at[i, pl.ds(HC, HC)]],
            rbufs[b].at[pl.ds(HC, HC)], gsems[b])

    def wait_gather(i, b):
        for h in (0, HC):
            pltpu.make_async_copy(
                x_hbm.at[idx_s.at[i, pl.ds(h, HC)]],
                rbufs[b].at[pl.ds(h, HC)], gsems[b]).wait()

    def fire_scatter(i, b):
        pltpu.async_copy(rbufs[b], acc.at[idx_d.at[i]], csems[b], add=True)

    def wait_scatter(i, b):
        pltpu.make_async_copy(rbufs[b], acc.at[idx_d.at[i]], csems[b]).wait()

    fire_gather(0, 0)
    fire_gather(1, 1)
    fire_gather(2, 2)
    wait_gather(0, 0)
    fire_scatter(0, 0)
    fire_gather(3, 3)
    wait_gather(1, 1)
    fire_scatter(1, 1)
    wait_scatter(0, 0)
    fire_gather(4, 0)
    wait_gather(2, 2)
    fire_scatter(2, 2)
    wait_scatter(1, 1)
    fire_gather(5, 1)
    wait_gather(3, 3)
    fire_scatter(3, 3)

    def group(g, carry):
        i0 = 4 * g
        for b in range(NB):
            i = i0 + b
            nb = (b + 2) % NB

            @pl.when(i + 2 < NCHUNKP)
            def _():
                wait_scatter(i - 2, nb)
                fire_gather(i + 2, nb)

            wait_gather(i, b)
            fire_scatter(i, b)
        return carry

    lax.fori_loop(1, NCHUNKP // NB, group, 0)

    for i in range(NCHUNKP - NB, NCHUNKP):
        wait_scatter(i, i % NB)
    plsc.subcore_barrier()

    def wblk(j, carry):
        r0 = (s + j * NS) * WB
        pltpu.sync_copy(acc.at[pl.ds(r0, WB)], o_hbm.at[pl.ds(r0, WB)])
        return carry

    lax.fori_loop(0, (140 - s) // 16, wblk, 0)
    plsc.subcore_barrier()


def _body(xa, xb, d1, s1, d2, s2, o1a, o1b, o2a, o2b,
          idx_d, idx_s, r0, r1, r2, r3, zbuf, acc,
          g0, g1, g2, g3, c0, c1, c2, c3):
    c = lax.axis_index("c")
    s = lax.axis_index("s")
    rbufs = (r0, r1, r2, r3)
    gsems = (g0, g1, g2, g3)
    csems = (c0, c1, c2, c3)

    def zrow(i, carry):
        for k in range(DH // 16):
            zbuf[i, pl.ds(k * 16, 16)] = jnp.zeros((16,), jnp.float32)
        return carry

    lax.fori_loop(0, ZB, zrow, 0)

    @pl.when(c == 0)
    def _():
        pltpu.sync_copy(d1.at[pl.ds(s * NCHUNKP, NCHUNKP)], idx_d)
        pltpu.sync_copy(s1.at[pl.ds(s * NCHUNKP, NCHUNKP)], idx_s)
        _hop_pass(xa, o1a, idx_d, idx_s, rbufs, zbuf, acc, gsems, csems, s)
        _hop_pass(xb, o1b, idx_d, idx_s, rbufs, zbuf, acc, gsems, csems, s)

    @pl.when(c == 1)
    def _():
        pltpu.sync_copy(d2.at[pl.ds(s * NCHUNKP, NCHUNKP)], idx_d)
        pltpu.sync_copy(s2.at[pl.ds(s * NCHUNKP, NCHUNKP)], idx_s)
        _hop_pass(xa, o2a, idx_d, idx_s, rbufs, zbuf, acc, gsems, csems, s)
        _hop_pass(xb, o2b, idx_d, idx_s, rbufs, zbuf, acc, gsems, csems, s)


_half_out = jax.ShapeDtypeStruct((N_NODES, DH), jnp.float32)

_sc_kernel = functools.partial(
    pl.kernel,
    mesh=plsc.VectorSubcoreMesh(core_axis_name="c", subcore_axis_name="s"),
    out_type=[_half_out, _half_out, _half_out, _half_out],
    compiler_params=pltpu.CompilerParams(use_tc_tiling_on_sc=False),
    scratch_types=[
        pltpu.VMEM((NCHUNKP, CS), jnp.int32),
        pltpu.VMEM((NCHUNKP, CS), jnp.int32),
        pltpu.VMEM((CS, DH), jnp.float32),
        pltpu.VMEM((CS, DH), jnp.float32),
        pltpu.VMEM((CS, DH), jnp.float32),
        pltpu.VMEM((CS, DH), jnp.float32),
        pltpu.VMEM((ZB, DH), jnp.float32),
        pltpu.VMEM_SHARED((ACC_ROWS, DH), jnp.float32),
        pltpu.SemaphoreType.DMA,
        pltpu.SemaphoreType.DMA,
        pltpu.SemaphoreType.DMA,
        pltpu.SemaphoreType.DMA,
        pltpu.SemaphoreType.DMA,
        pltpu.SemaphoreType.DMA,
        pltpu.SemaphoreType.DMA,
        pltpu.SemaphoreType.DMA,
    ],
)(_body)


def _pad_idx(row, pad_val):
    pad = jnp.full((EPAD - E,), pad_val, jnp.int32)
    return jnp.concatenate([row, pad]).reshape(NS * NCHUNKP, CS)


@jax.jit
def kernel(x, adj_t, adj_t2):
    xa = x[:, :DH]
    xb = x[:, DH:]
    d1 = _pad_idx(adj_t[0], GARB)
    s1 = _pad_idx(adj_t[1], 0)
    d2 = _pad_idx(adj_t2[0], GARB)
    s2 = _pad_idx(adj_t2[1], 0)
    o1a, o1b, o2a, o2b = _sc_kernel(xa, xb, d1, s1, d2, s2)
    return jnp.concatenate([o1a, o1b, o2a, o2b], axis=1)


# speedup vs baseline: 9.4125x; 2.5200x over previous
import functools

import jax
import jax.numpy as jnp
from jax import lax
from jax.experimental import pallas as pl
from jax.experimental.pallas import tpu as pltpu
from jax.experimental.pallas import tpu_sc as plsc

N_NODES = 10000
D = 128
DH = D // 2
E = 320000
NS = 16
CS = 128
NCHUNKP = 160
EPAD = NS * NCHUNKP * CS
GARB = N_NODES
ACC_ROWS = 10016
ZB = 16
WB = 80
NB = 4
NBI = 8


def _hop_pass(x_hbm, o_hbm, d_hbm, s_hbm, xspm, isd, iss, rbufs, zbuf, acc,
              isems, gsems, csems, s, row0):
    def xblk(j, carry):
        r0 = (s + j * NS) * WB
        pltpu.sync_copy(x_hbm.at[pl.ds(r0, WB)], xspm.at[pl.ds(r0, WB)])
        return carry

    lax.fori_loop(0, (140 - s) // 16, xblk, 0)

    def zblk(j, carry):
        pltpu.sync_copy(zbuf, acc.at[pl.ds((s + j * NS) * ZB, ZB)])
        return carry

    lax.fori_loop(0, (641 - s) // 16, zblk, 0)
    plsc.subcore_barrier()

    def fire_idx(i, t):
        pltpu.async_copy(d_hbm.at[row0 + i], isd.at[t], isems[t])
        pltpu.async_copy(s_hbm.at[row0 + i], iss.at[t], isems[t])

    def wait_idx(i, t):
        pltpu.make_async_copy(d_hbm.at[row0 + i], isd.at[t], isems[t]).wait()
        pltpu.make_async_copy(s_hbm.at[row0 + i], iss.at[t], isems[t]).wait()

    def fire_gather(i, b, t):
        pltpu.async_copy(xspm.at[iss.at[t]], rbufs[b], gsems[b])

    def wait_gather(i, b, t):
        pltpu.make_async_copy(xspm.at[iss.at[t]], rbufs[b], gsems[b]).wait()

    def fire_scatter(i, b, t):
        pltpu.async_copy(rbufs[b], acc.at[isd.at[t]], csems[b], add=True)

    def wait_scatter(i, b, t):
        pltpu.make_async_copy(rbufs[b], acc.at[isd.at[t]], csems[b]).wait()

    for i in range(4):
        fire_idx(i, i)
    wait_idx(0, 0)
    fire_gather(0, 0, 0)
    wait_idx(1, 1)
    fire_gather(1, 1, 1)

    def do_step(i, ring_i, ring_i2, slot_i, slot_i2, slot_i4,
                skip_scatter_wait):
        @pl.when(i + 2 < NCHUNKP)
        def _():
            if not skip_scatter_wait:
                wait_scatter(i - 2, ring_i2, (slot_i2 + NBI - 4) % NBI)
            wait_idx(i + 2, slot_i2)
            fire_gather(i + 2, ring_i2, slot_i2)

        @pl.when(i + 4 < NCHUNKP)
        def _():
            fire_idx(i + 4, slot_i4)

        wait_gather(i, ring_i, slot_i)
        fire_scatter(i, ring_i, slot_i)

    for b in range(NBI):
        do_step(b, b % NB, (b + 2) % NB, b % NBI, (b + 2) % NBI,
                (b + 4) % NBI, b < 2)

    def group(g, carry):
        i0 = NBI * g
        for b in range(NBI):
            i = i0 + b
            do_step(i, b % NB, (b + 2) % NB, b % NBI, (b + 2) % NBI,
                    (b + 4) % NBI, False)
        return carry

    lax.fori_loop(1, NCHUNKP // NBI, group, 0)

    for i in range(NCHUNKP - NB, NCHUNKP):
        wait_scatter(i, i % NB, i % NBI)
    plsc.subcore_barrier()

    def wblk(j, carry):
        r0 = (s + j * NS) * WB
        pltpu.sync_copy(acc.at[pl.ds(r0, WB)], o_hbm.at[pl.ds(r0, WB)])
        return carry

    lax.fori_loop(0, (140 - s) // 16, wblk, 0)
    plsc.subcore_barrier()


def _body(xa, xb, d1, s1, d2, s2, o1a, o1b, o2a, o2b,
          isd, iss, r0, r1, r2, r3, zbuf, xspm, acc,
          i0, i1, i2, i3, i4, i5, i6, i7,
          g0, g1, g2, g3, c0, c1, c2, c3):
    c = lax.axis_index("c")
    s = lax.axis_index("s")
    rbufs = (r0, r1, r2, r3)
    isems = (i0, i1, i2, i3, i4, i5, i6, i7)
    gsems = (g0, g1, g2, g3)
    csems = (c0, c1, c2, c3)
    row0 = s * NCHUNKP

    def zrow(i, carry):
        for k in range(DH // 16):
            zbuf[i, pl.ds(k * 16, 16)] = jnp.zeros((16,), jnp.float32)
        return carry

    lax.fori_loop(0, ZB, zrow, 0)

    @pl.when(c == 0)
    def _():
        _hop_pass(xa, o1a, d1, s1, xspm, isd, iss, rbufs, zbuf, acc,
                  isems, gsems, csems, s, row0)
        _hop_pass(xb, o1b, d1, s1, xspm, isd, iss, rbufs, zbuf, acc,
                  isems, gsems, csems, s, row0)

    @pl.when(c == 1)
    def _():
        _hop_pass(xa, o2a, d2, s2, xspm, isd, iss, rbufs, zbuf, acc,
                  isems, gsems, csems, s, row0)
        _hop_pass(xb, o2b, d2, s2, xspm, isd, iss, rbufs, zbuf, acc,
                  isems, gsems, csems, s, row0)


_half_out = jax.ShapeDtypeStruct((N_NODES, DH), jnp.float32)

_sc_kernel = functools.partial(
    pl.kernel,
    mesh=plsc.VectorSubcoreMesh(core_axis_name="c", subcore_axis_name="s"),
    out_type=[_half_out, _half_out, _half_out, _half_out],
    compiler_params=pltpu.CompilerParams(use_tc_tiling_on_sc=False),
    scratch_types=[
        pltpu.VMEM((NBI, CS), jnp.int32),
        pltpu.VMEM((NBI, CS), jnp.int32),
    ] + [pltpu.VMEM((CS, DH), jnp.float32)] * NB
    + [
        pltpu.VMEM((ZB, DH), jnp.float32),
        pltpu.VMEM_SHARED((N_NODES, DH), jnp.float32),
        pltpu.VMEM_SHARED((ACC_ROWS, DH), jnp.float32),
    ] + [pltpu.SemaphoreType.DMA] * (NBI + 2 * NB),
)(_body)


def _pad_idx(row, pad_val):
    pad = jnp.full((EPAD - E,), pad_val, jnp.int32)
    return jnp.concatenate([row, pad]).reshape(NS * NCHUNKP, CS)


@jax.jit
def kernel(x, adj_t, adj_t2):
    xa = x[:, :DH]
    xb = x[:, DH:]
    d1 = _pad_idx(adj_t[0], GARB)
    s1 = _pad_idx(adj_t[1], 0)
    d2 = _pad_idx(adj_t2[0], GARB)
    s2 = _pad_idx(adj_t2[1], 0)
    o1a, o1b, o2a, o2b = _sc_kernel(xa, xb, d1, s1, d2, s2)
    return jnp.concatenate([o1a, o1b, o2a, o2b], axis=1)
